# baseline (speedup 1.0000x reference)
import functools

import jax
import jax.numpy as jnp
from jax import lax
from jax.experimental import pallas as pl
from jax.experimental.pallas import tpu as pltpu
from jax.experimental.pallas import tpu_sc as plsc

_B = 512
_T = 128
_D = 1024
_PP = 4096
_P = 2 * _PP
_L = 5
_K = 8
_OUT_ROWS = _K * _L + _T

_NC, _NS = 2, 16
_NW = _NC * _NS
_BPW = _B // _NW
_PAIRS = _BPW // 2


def _mean_norm_body(x_ref, q_ref):
    m = jnp.mean(x_ref[...], axis=1)
    ss = jnp.sum(m * m, axis=1, keepdims=True)
    q_ref[...] = m * lax.rsqrt(jnp.maximum(ss, 1e-12))


def _sim_body(q_ref, k_ref, s_ref):
    k = k_ref[...]
    ss = jnp.sum(k * k, axis=1, keepdims=True)
    kn = k * lax.rsqrt(jnp.maximum(ss, 1e-12))
    s_ref[...] = lax.dot_general(
        q_ref[...], kn, (((1,), (1,)), ((), ())),
        preferred_element_type=jnp.float32)


def _topk_body(s_ref, idx_ref, row_ref, acc_ref):
    val = s_ref[...]
    it = lax.broadcasted_iota(jnp.int32, val.shape, 1)
    cols = []
    tot = jnp.float32(0.0)
    for _ in range(_K):
        m = jnp.max(val, axis=1, keepdims=True)
        sel = jnp.min(jnp.where(val == m, it, _P), axis=1, keepdims=True)
        cols.append(sel)
        tot = tot + jnp.sum(m)
        val = jnp.where(it == sel, -jnp.inf, val)
    idx_ref[...] = jnp.concatenate(cols, axis=1)
    row_ref[...] = jnp.concatenate(
        [cols[k] * _L + j for k in range(_K) for j in range(_L)], axis=1)
    acc_ref[...] = jnp.full((1, 1, 128), tot, jnp.float32)


def _sc_body(prompt_hbm, row_hbm, x_hbm, out_hbm, rowidx, rows, xbuf, sem):
    wid = lax.axis_index("s") * _NC + lax.axis_index("c")

    def pair_body(i, carry):
        b0 = wid * _BPW + i * 2
        pltpu.sync_copy(row_hbm.at[pl.ds(b0 * _K * _L, 2 * _K * _L)], rowidx)
        pltpu.async_copy(prompt_hbm.at[rowidx], rows, sem).wait()
        pltpu.sync_copy(rows.at[pl.ds(0, _K * _L)],
                        out_hbm.at[pl.ds(b0 * _OUT_ROWS, _K * _L)])
        pltpu.sync_copy(rows.at[pl.ds(_K * _L, _K * _L)],
                        out_hbm.at[pl.ds((b0 + 1) * _OUT_ROWS, _K * _L)])
        return carry

    lax.fori_loop(0, _PAIRS, pair_body, 0)

    def copy_body(i, carry):
        b = wid * _BPW + i
        pltpu.sync_copy(x_hbm.at[pl.ds(b * _T, _T)],
                        out_hbm.at[pl.ds(b * _OUT_ROWS + _K * _L, _T)])
        return carry

    lax.fori_loop(0, _BPW, copy_body, 0)


@functools.cache
def _sc_gather():
    return pl.kernel(
        _sc_body,
        out_type=jax.ShapeDtypeStruct((_B * _OUT_ROWS, _D), jnp.float32),
        mesh=plsc.VectorSubcoreMesh(core_axis_name="c", subcore_axis_name="s",
                                    num_cores=_NC, num_subcores=_NS),
        scratch_types=[
            pltpu.VMEM((2 * _K * _L,), jnp.int32),
            pltpu.VMEM((2 * _K * _L, _D), jnp.float32),
            pltpu.VMEM((32, _D), jnp.float32),
            pltpu.SemaphoreType.DMA,
        ],
    )


@jax.jit
def kernel(x_embed, prompt_old, prompt_new, prompt_key_old, prompt_key_new):
    keys = jnp.concatenate([prompt_key_old, prompt_key_new], axis=0)
    prompt2d = jnp.concatenate([prompt_old, prompt_new], axis=0)
    prompt2d = prompt2d.reshape(_P * _L, _D)

    q_norm = pl.pallas_call(
        _mean_norm_body,
        grid=(16,),
        in_specs=[pl.BlockSpec((_B // 16, _T, _D), lambda i: (i, 0, 0))],
        out_specs=pl.BlockSpec((_B // 16, _D), lambda i: (i, 0)),
        out_shape=jax.ShapeDtypeStruct((_B, _D), jnp.float32),
    )(x_embed)

    sim = pl.pallas_call(
        _sim_body,
        grid=(8,),
        in_specs=[
            pl.BlockSpec((_B, _D), lambda i: (0, 0)),
            pl.BlockSpec((_P // 8, _D), lambda i: (i, 0)),
        ],
        out_specs=pl.BlockSpec((_B, _P // 8), lambda i: (0, i)),
        out_shape=jax.ShapeDtypeStruct((_B, _P), jnp.float32),
    )(q_norm, keys)

    idx, row, acc = pl.pallas_call(
        _topk_body,
        grid=(4,),
        in_specs=[pl.BlockSpec((_B // 4, _P), lambda i: (i, 0))],
        out_specs=[
            pl.BlockSpec((_B // 4, _K), lambda i: (i, 0)),
            pl.BlockSpec((_B // 4, _K * _L), lambda i: (i, 0)),
            pl.BlockSpec((1, 1, 128), lambda i: (i, 0, 0)),
        ],
        out_shape=[
            jax.ShapeDtypeStruct((_B, _K), jnp.int32),
            jax.ShapeDtypeStruct((_B, _K * _L), jnp.int32),
            jax.ShapeDtypeStruct((4, 1, 128), jnp.float32),
        ],
    )(sim)

    out2d = _sc_gather()(prompt2d, row.reshape(-1),
                         x_embed.reshape(_B * _T, _D))
    prompted = out2d.reshape(_B, _OUT_ROWS, _D)
    reduce_sim = jnp.sum(acc[:, 0, 0]) / _B
    return prompted, reduce_sim, sim, idx


# speedup vs baseline: 6.0835x; 6.0835x over previous
import functools

import jax
import jax.numpy as jnp
from jax import lax
from jax.experimental import pallas as pl
from jax.experimental.pallas import tpu as pltpu
from jax.experimental.pallas import tpu_sc as plsc

_B = 512
_T = 128
_D = 1024
_PP = 4096
_P = 2 * _PP
_L = 5
_K = 8
_OUT_ROWS = _K * _L + _T

_NC, _NS = 2, 16
_NW = _NC * _NS
_BPW = _B // _NW
_PAIRS = _BPW // 2


def _mean_norm_body(x_ref, q_ref):
    m = jnp.mean(x_ref[...], axis=1)
    ss = jnp.sum(m * m, axis=1, keepdims=True)
    q_ref[...] = m * lax.rsqrt(jnp.maximum(ss, 1e-12))


def _sim_body(q_ref, k_ref, s_ref):
    k = k_ref[...]
    ss = jnp.sum(k * k, axis=1, keepdims=True)
    kn = k * lax.rsqrt(jnp.maximum(ss, 1e-12))
    s_ref[...] = lax.dot_general(
        q_ref[...], kn, (((1,), (1,)), ((), ())),
        preferred_element_type=jnp.float32)


_KL = _K * _L


def _topk_body(s_ref, idx_ref, gsrc_ref, gdst_ref, acc_ref):
    val = s_ref[...]
    it = lax.broadcasted_iota(jnp.int32, val.shape, 1)
    bb = val.shape[0]
    brow = (lax.broadcasted_iota(jnp.int32, (bb, 1), 0)
            + pl.program_id(0) * bb)
    cols = []
    tot = jnp.float32(0.0)
    for _ in range(_K):
        m = jnp.max(val, axis=1, keepdims=True)
        sel = jnp.min(jnp.where(val == m, it, _P), axis=1, keepdims=True)
        cols.append(sel)
        tot = tot + jnp.sum(m)
        val = jnp.where(it == sel, -jnp.inf, val)
    idx_ref[...] = jnp.concatenate(cols, axis=1)
    so, sn, do_, dn = [], [], [], []
    for k in range(_K):
        for j in range(_L):
            p = _L * k + j
            row = cols[k] * _L + j
            old = cols[k] < _PP
            so.append(jnp.minimum(row, _PP * _L - 1))
            sn.append(jnp.maximum(row - _PP * _L, 0))
            real = brow * _OUT_ROWS + p
            junk = brow * _OUT_ROWS + _KL + (p % 8)
            do_.append(jnp.where(old, real, junk))
            dn.append(jnp.where(old, junk, real))
    gsrc_ref[...] = jnp.concatenate(so + sn, axis=1)
    gdst_ref[...] = jnp.concatenate(do_ + dn, axis=1)
    acc_ref[...] = jnp.full((1, 1, 128), tot, jnp.float32)


_XCH = 16
_NCHB = _T // _XCH


def _sc_body(pold_hbm, pnew_hbm, gsrc_hbm, gdst_hbm, x_hbm, out_hbm,
             gsrcv, gdstv, rold, rnew, xb0, xb1,
             sgo, sgn, sso, ssn, rs0, rs1, ws0, ws1):
    wid = lax.axis_index("s") * _NC + lax.axis_index("c")
    base = wid * _BPW

    for b in range(_BPW):
        bb = base + b
        pltpu.sync_copy(gsrc_hbm.at[bb], gsrcv)
        pltpu.sync_copy(gdst_hbm.at[bb], gdstv)
        h1 = pltpu.async_copy(pold_hbm.at[gsrcv.at[pl.ds(0, _KL)]], rold, sgo)
        h2 = pltpu.async_copy(pnew_hbm.at[gsrcv.at[pl.ds(_KL, _KL)]], rnew,
                              sgn)
        h1.wait()
        h2.wait()
        w1 = pltpu.async_copy(rold, out_hbm.at[gdstv.at[0]], sso)
        w2 = pltpu.async_copy(rnew, out_hbm.at[gdstv.at[1]], ssn)
        w1.wait()
        w2.wait()

    xb = (xb0, xb1)
    rsem = (rs0, rs1)
    wsem = (ws0, ws1)
    nch = _BPW * _NCHB

    def xs(c):
        return (base + c // _NCHB) * _T + (c % _NCHB) * _XCH

    def xd(c):
        return (base + c // _NCHB) * _OUT_ROWS + _KL + (c % _NCHB) * _XCH

    rh = [pltpu.async_copy(x_hbm.at[pl.ds(xs(0), _XCH)], xb0, rs0), None]
    wh = [None, None]
    for c in range(nch):
        slot = c % 2
        rh[slot].wait()
        if c + 1 < nch:
            nxt = (c + 1) % 2
            if wh[nxt] is not None:
                wh[nxt].wait()
            rh[nxt] = pltpu.async_copy(
                x_hbm.at[pl.ds(xs(c + 1), _XCH)], xb[nxt], rsem[nxt])
        wh[slot] = pltpu.async_copy(
            xb[slot], out_hbm.at[pl.ds(xd(c), _XCH)], wsem[slot])
    wh[0].wait()
    wh[1].wait()


@functools.cache
def _sc_gather():
    return pl.kernel(
        _sc_body,
        out_type=jax.ShapeDtypeStruct((_B * _OUT_ROWS, _D), jnp.float32),
        mesh=plsc.VectorSubcoreMesh(core_axis_name="c", subcore_axis_name="s",
                                    num_cores=_NC, num_subcores=_NS),
        scratch_types=[
            pltpu.VMEM((2 * _KL,), jnp.int32),
            pltpu.VMEM((2, _KL), jnp.int32),
            pltpu.VMEM((_KL, _D), jnp.float32),
            pltpu.VMEM((_KL, _D), jnp.float32),
            pltpu.VMEM((_XCH, _D), jnp.float32),
            pltpu.VMEM((_XCH, _D), jnp.float32),
            pltpu.SemaphoreType.DMA,
            pltpu.SemaphoreType.DMA,
            pltpu.SemaphoreType.DMA,
            pltpu.SemaphoreType.DMA,
            pltpu.SemaphoreType.DMA,
            pltpu.SemaphoreType.DMA,
            pltpu.SemaphoreType.DMA,
            pltpu.SemaphoreType.DMA,
        ],
    )


@jax.jit
def kernel(x_embed, prompt_old, prompt_new, prompt_key_old, prompt_key_new):
    keys = jnp.concatenate([prompt_key_old, prompt_key_new], axis=0)
    pold2d = prompt_old.reshape(_PP * _L, _D)
    pnew2d = prompt_new.reshape(_PP * _L, _D)

    q_norm = pl.pallas_call(
        _mean_norm_body,
        grid=(16,),
        in_specs=[pl.BlockSpec((_B // 16, _T, _D), lambda i: (i, 0, 0))],
        out_specs=pl.BlockSpec((_B // 16, _D), lambda i: (i, 0)),
        out_shape=jax.ShapeDtypeStruct((_B, _D), jnp.float32),
    )(x_embed)

    sim = pl.pallas_call(
        _sim_body,
        grid=(8,),
        in_specs=[
            pl.BlockSpec((_B, _D), lambda i: (0, 0)),
            pl.BlockSpec((_P // 8, _D), lambda i: (i, 0)),
        ],
        out_specs=pl.BlockSpec((_B, _P // 8), lambda i: (0, i)),
        out_shape=jax.ShapeDtypeStruct((_B, _P), jnp.float32),
    )(q_norm, keys)

    idx, gsrc, gdst, acc = pl.pallas_call(
        _topk_body,
        grid=(4,),
        in_specs=[pl.BlockSpec((_B // 4, _P), lambda i: (i, 0))],
        out_specs=[
            pl.BlockSpec((_B // 4, _K), lambda i: (i, 0)),
            pl.BlockSpec((_B // 4, 2 * _KL), lambda i: (i, 0)),
            pl.BlockSpec((_B // 4, 2 * _KL), lambda i: (i, 0)),
            pl.BlockSpec((1, 1, 128), lambda i: (i, 0, 0)),
        ],
        out_shape=[
            jax.ShapeDtypeStruct((_B, _K), jnp.int32),
            jax.ShapeDtypeStruct((_B, 2 * _KL), jnp.int32),
            jax.ShapeDtypeStruct((_B, 2 * _KL), jnp.int32),
            jax.ShapeDtypeStruct((4, 1, 128), jnp.float32),
        ],
    )(sim)

    out2d = _sc_gather()(pold2d, pnew2d, gsrc, gdst.reshape(_B, 2, _KL),
                         x_embed.reshape(_B * _T, _D))
    prompted = out2d.reshape(_B, _OUT_ROWS, _D)
    reduce_sim = jnp.sum(acc[:, 0, 0]) / _B
    return prompted, reduce_sim, sim, idx


# speedup vs baseline: 11.0746x; 1.8204x over previous
import functools

import jax
import jax.numpy as jnp
from jax import lax
from jax.experimental import pallas as pl
from jax.experimental.pallas import tpu as pltpu
from jax.experimental.pallas import tpu_sc as plsc

_B = 512
_T = 128
_D = 1024
_PP = 4096
_P = 2 * _PP
_L = 5
_K = 8
_OUT_ROWS = _K * _L + _T

_NC, _NS = 2, 16
_NW = _NC * _NS
_BPW = _B // _NW
_PAIRS = _BPW // 2


_NCAT = 32


def _concat_body(old_ref, new_ref, out_ref):
    i = pl.program_id(0)
    blk = jnp.where(i < _NCAT // 2, old_ref[...], new_ref[...])
    out_ref[...] = blk.reshape(out_ref.shape)


def _mean_norm_body(x_ref, q_ref):
    m = jnp.mean(x_ref[...], axis=1)
    ss = jnp.sum(m * m, axis=1, keepdims=True)
    q_ref[...] = m * lax.rsqrt(jnp.maximum(ss, 1e-12))


def _sim_body(q_ref, k_ref, s_ref):
    k = k_ref[...]
    ss = jnp.sum(k * k, axis=1, keepdims=True)
    kn = k * lax.rsqrt(jnp.maximum(ss, 1e-12))
    s_ref[...] = lax.dot_general(
        q_ref[...], kn, (((1,), (1,)), ((), ())),
        preferred_element_type=jnp.float32)


_KL = _K * _L


def _topk_body(s_ref, idx_ref, row_ref, acc_ref):
    val = s_ref[...]
    it = lax.broadcasted_iota(jnp.int32, val.shape, 1)
    cols = []
    tot = jnp.float32(0.0)
    for _ in range(_K):
        m = jnp.max(val, axis=1, keepdims=True)
        sel = jnp.min(jnp.where(val == m, it, _P), axis=1, keepdims=True)
        cols.append(sel)
        tot = tot + jnp.sum(m)
        val = jnp.where(it == sel, -jnp.inf, val)
    idx_ref[...] = jnp.concatenate(cols, axis=1)
    row_ref[...] = jnp.concatenate(
        [cols[k] * _L + j for k in range(_K) for j in range(_L)], axis=1)
    acc_ref[...] = jnp.full((1, 1, 128), tot, jnp.float32)


_XCH = 16
_NCHB = _T // _XCH


def _sc_body(prompt_hbm, row_hbm, x_hbm, out_hbm,
             ri0, ri1, g0, g1, xb0, xb1,
             gs0, gs1, gw0, gw1, rs0, rs1, ws0, ws1):
    wid = lax.axis_index("s") * _NC + lax.axis_index("c")
    base = wid * _BPW

    ri = (ri0, ri1)
    gb = (g0, g1)
    gsem = (gs0, gs1)
    gwsem = (gw0, gw1)

    def gstart(b, slot):
        pltpu.sync_copy(row_hbm.at[pl.ds((base + b) * _KL, _KL)], ri[slot])
        return pltpu.async_copy(prompt_hbm.at[ri[slot]], gb[slot], gsem[slot])

    gh = [gstart(0, 0), None]
    gwh = [None, None]
    for b in range(_BPW):
        slot = b % 2
        gh[slot].wait()
        if b + 1 < _BPW:
            nxt = (b + 1) % 2
            if gwh[nxt] is not None:
                gwh[nxt].wait()
            gh[nxt] = gstart(b + 1, nxt)
        gwh[slot] = pltpu.async_copy(
            gb[slot], out_hbm.at[pl.ds((base + b) * _OUT_ROWS, _KL)],
            gwsem[slot])
    gwh[0].wait()
    gwh[1].wait()

    xb = (xb0, xb1)
    rsem = (rs0, rs1)
    wsem = (ws0, ws1)
    nch = _BPW * _NCHB

    def xs(c):
        return (base + c // _NCHB) * _T + (c % _NCHB) * _XCH

    def xd(c):
        return (base + c // _NCHB) * _OUT_ROWS + _KL + (c % _NCHB) * _XCH

    rh = [pltpu.async_copy(x_hbm.at[pl.ds(xs(0), _XCH)], xb0, rs0), None]
    wh = [None, None]
    for c in range(nch):
        slot = c % 2
        rh[slot].wait()
        if c + 1 < nch:
            nxt = (c + 1) % 2
            if wh[nxt] is not None:
                wh[nxt].wait()
            rh[nxt] = pltpu.async_copy(
                x_hbm.at[pl.ds(xs(c + 1), _XCH)], xb[nxt], rsem[nxt])
        wh[slot] = pltpu.async_copy(
            xb[slot], out_hbm.at[pl.ds(xd(c), _XCH)], wsem[slot])
    wh[0].wait()
    wh[1].wait()


@functools.cache
def _sc_gather():
    return pl.kernel(
        _sc_body,
        out_type=jax.ShapeDtypeStruct((_B * _OUT_ROWS, _D), jnp.float32),
        mesh=plsc.VectorSubcoreMesh(core_axis_name="c", subcore_axis_name="s",
                                    num_cores=_NC, num_subcores=_NS),
        scratch_types=[
            pltpu.VMEM((_KL,), jnp.int32),
            pltpu.VMEM((_KL,), jnp.int32),
            pltpu.VMEM((_KL, _D), jnp.float32),
            pltpu.VMEM((_KL, _D), jnp.float32),
            pltpu.VMEM((_XCH, _D), jnp.float32),
            pltpu.VMEM((_XCH, _D), jnp.float32),
            pltpu.SemaphoreType.DMA,
            pltpu.SemaphoreType.DMA,
            pltpu.SemaphoreType.DMA,
            pltpu.SemaphoreType.DMA,
            pltpu.SemaphoreType.DMA,
            pltpu.SemaphoreType.DMA,
            pltpu.SemaphoreType.DMA,
            pltpu.SemaphoreType.DMA,
        ],
    )


@jax.jit
def kernel(x_embed, prompt_old, prompt_new, prompt_key_old, prompt_key_new):
    keys = jnp.concatenate([prompt_key_old, prompt_key_new], axis=0)

    _half = _NCAT // 2
    _rpb = _PP // _half
    prompt2d = pl.pallas_call(
        _concat_body,
        grid=(_NCAT,),
        in_specs=[
            pl.BlockSpec((_rpb, _L, _D),
                         lambda i: (jnp.minimum(i, _half - 1), 0, 0)),
            pl.BlockSpec((_rpb, _L, _D),
                         lambda i: (jnp.maximum(i - _half, 0), 0, 0)),
        ],
        out_specs=pl.BlockSpec((_rpb * _L, _D), lambda i: (i, 0)),
        out_shape=jax.ShapeDtypeStruct((_P * _L, _D), jnp.float32),
    )(prompt_old, prompt_new)

    q_norm = pl.pallas_call(
        _mean_norm_body,
        grid=(16,),
        in_specs=[pl.BlockSpec((_B // 16, _T, _D), lambda i: (i, 0, 0))],
        out_specs=pl.BlockSpec((_B // 16, _D), lambda i: (i, 0)),
        out_shape=jax.ShapeDtypeStruct((_B, _D), jnp.float32),
    )(x_embed)

    sim = pl.pallas_call(
        _sim_body,
        grid=(8,),
        in_specs=[
            pl.BlockSpec((_B, _D), lambda i: (0, 0)),
            pl.BlockSpec((_P // 8, _D), lambda i: (i, 0)),
        ],
        out_specs=pl.BlockSpec((_B, _P // 8), lambda i: (0, i)),
        out_shape=jax.ShapeDtypeStruct((_B, _P), jnp.float32),
    )(q_norm, keys)

    idx, row, acc = pl.pallas_call(
        _topk_body,
        grid=(4,),
        in_specs=[pl.BlockSpec((_B // 4, _P), lambda i: (i, 0))],
        out_specs=[
            pl.BlockSpec((_B // 4, _K), lambda i: (i, 0)),
            pl.BlockSpec((_B // 4, _KL), lambda i: (i, 0)),
            pl.BlockSpec((1, 1, 128), lambda i: (i, 0, 0)),
        ],
        out_shape=[
            jax.ShapeDtypeStruct((_B, _K), jnp.int32),
            jax.ShapeDtypeStruct((_B, _KL), jnp.int32),
            jax.ShapeDtypeStruct((4, 1, 128), jnp.float32),
        ],
    )(sim)

    out2d = _sc_gather()(prompt2d, row.reshape(-1),
                         x_embed.reshape(_B * _T, _D))
    prompted = out2d.reshape(_B, _OUT_ROWS, _D)
    reduce_sim = jnp.sum(acc[:, 0, 0]) / _B
    return prompted, reduce_sim, sim, idx


# speedup vs baseline: 11.3938x; 1.0288x over previous
import functools

import jax
import jax.numpy as jnp
from jax import lax
from jax.experimental import pallas as pl
from jax.experimental.pallas import tpu as pltpu
from jax.experimental.pallas import tpu_sc as plsc

_B = 512
_T = 128
_D = 1024
_PP = 4096
_P = 2 * _PP
_L = 5
_K = 8
_OUT_ROWS = _K * _L + _T

_NC, _NS = 2, 16
_NW = _NC * _NS
_BPW = _B // _NW
_PAIRS = _BPW // 2


_NCAT = 32


def _concat_body(old_ref, new_ref, out_ref):
    i = pl.program_id(0)
    blk = jnp.where(i < _NCAT // 2, old_ref[...], new_ref[...])
    out_ref[...] = blk.reshape(out_ref.shape)


def _mean_norm_body(x_ref, q_ref):
    m = jnp.mean(x_ref[...], axis=1)
    ss = jnp.sum(m * m, axis=1, keepdims=True)
    q_ref[...] = m * lax.rsqrt(jnp.maximum(ss, 1e-12))


def _sim_body(q_ref, ko_ref, kn_ref, s_ref):
    i = pl.program_id(0)
    k = jnp.where(i < 4, ko_ref[...], kn_ref[...])
    ss = jnp.sum(k * k, axis=1, keepdims=True)
    kn = k * lax.rsqrt(jnp.maximum(ss, 1e-12))
    s_ref[...] = lax.dot_general(
        q_ref[...], kn, (((1,), (1,)), ((), ())),
        preferred_element_type=jnp.float32)


_KL = _K * _L


def _topk_body(s_ref, idx_ref, row_ref, acc_ref):
    val = s_ref[...]
    it = lax.broadcasted_iota(jnp.int32, val.shape, 1)
    cols = []
    tot = jnp.float32(0.0)
    for _ in range(_K):
        m = jnp.max(val, axis=1, keepdims=True)
        sel = jnp.min(jnp.where(val == m, it, _P), axis=1, keepdims=True)
        cols.append(sel)
        tot = tot + jnp.sum(m)
        val = jnp.where(it == sel, -jnp.inf, val)
    idx_ref[...] = jnp.concatenate(cols, axis=1)
    row_ref[...] = jnp.concatenate(
        [cols[k] * _L + j for k in range(_K) for j in range(_L)], axis=1)
    acc_ref[...] = jnp.full((1, 1, 128), tot, jnp.float32)


_XCH = 16
_NCHB = _T // _XCH


def _sc_body(prompt_hbm, row_hbm, x_hbm, out_hbm,
             ri0, ri1, g0, g1, xb0, xb1,
             gs0, gs1, gw0, gw1, rs0, rs1, ws0, ws1):
    wid = lax.axis_index("s") * _NC + lax.axis_index("c")
    base = wid * _BPW

    ri = (ri0, ri1)
    gb = (g0, g1)
    gsem = (gs0, gs1)
    gwsem = (gw0, gw1)

    def gstart(b, slot):
        pltpu.sync_copy(row_hbm.at[pl.ds((base + b) * _KL, _KL)], ri[slot])
        return pltpu.async_copy(prompt_hbm.at[ri[slot]], gb[slot], gsem[slot])

    gh = [gstart(0, 0), None]
    gwh = [None, None]
    for b in range(_BPW):
        slot = b % 2
        gh[slot].wait()
        if b + 1 < _BPW:
            nxt = (b + 1) % 2
            if gwh[nxt] is not None:
                gwh[nxt].wait()
            gh[nxt] = gstart(b + 1, nxt)
        gwh[slot] = pltpu.async_copy(
            gb[slot], out_hbm.at[pl.ds((base + b) * _OUT_ROWS, _KL)],
            gwsem[slot])
    gwh[0].wait()
    gwh[1].wait()

    xb = (xb0, xb1)
    rsem = (rs0, rs1)
    wsem = (ws0, ws1)
    nch = _BPW * _NCHB

    def xs(c):
        return (base + c // _NCHB) * _T + (c % _NCHB) * _XCH

    def xd(c):
        return (base + c // _NCHB) * _OUT_ROWS + _KL + (c % _NCHB) * _XCH

    rh = [pltpu.async_copy(x_hbm.at[pl.ds(xs(0), _XCH)], xb0, rs0), None]
    wh = [None, None]
    for c in range(nch):
        slot = c % 2
        rh[slot].wait()
        if c + 1 < nch:
            nxt = (c + 1) % 2
            if wh[nxt] is not None:
                wh[nxt].wait()
            rh[nxt] = pltpu.async_copy(
                x_hbm.at[pl.ds(xs(c + 1), _XCH)], xb[nxt], rsem[nxt])
        wh[slot] = pltpu.async_copy(
            xb[slot], out_hbm.at[pl.ds(xd(c), _XCH)], wsem[slot])
    wh[0].wait()
    wh[1].wait()


@functools.cache
def _sc_gather():
    return pl.kernel(
        _sc_body,
        out_type=jax.ShapeDtypeStruct((_B * _OUT_ROWS, _D), jnp.float32),
        mesh=plsc.VectorSubcoreMesh(core_axis_name="c", subcore_axis_name="s",
                                    num_cores=_NC, num_subcores=_NS),
        scratch_types=[
            pltpu.VMEM((_KL,), jnp.int32),
            pltpu.VMEM((_KL,), jnp.int32),
            pltpu.VMEM((_KL, _D), jnp.float32),
            pltpu.VMEM((_KL, _D), jnp.float32),
            pltpu.VMEM((_XCH, _D), jnp.float32),
            pltpu.VMEM((_XCH, _D), jnp.float32),
            pltpu.SemaphoreType.DMA,
            pltpu.SemaphoreType.DMA,
            pltpu.SemaphoreType.DMA,
            pltpu.SemaphoreType.DMA,
            pltpu.SemaphoreType.DMA,
            pltpu.SemaphoreType.DMA,
            pltpu.SemaphoreType.DMA,
            pltpu.SemaphoreType.DMA,
        ],
    )


@jax.jit
def kernel(x_embed, prompt_old, prompt_new, prompt_key_old, prompt_key_new):
    _half = _NCAT // 2
    _rpb = _PP // _half
    prompt2d = pl.pallas_call(
        _concat_body,
        grid=(_NCAT,),
        in_specs=[
            pl.BlockSpec((_rpb, _L, _D),
                         lambda i: (jnp.minimum(i, _half - 1), 0, 0)),
            pl.BlockSpec((_rpb, _L, _D),
                         lambda i: (jnp.maximum(i - _half, 0), 0, 0)),
        ],
        out_specs=pl.BlockSpec((_rpb * _L, _D), lambda i: (i, 0)),
        out_shape=jax.ShapeDtypeStruct((_P * _L, _D), jnp.float32),
    )(prompt_old, prompt_new)

    q_norm = pl.pallas_call(
        _mean_norm_body,
        grid=(16,),
        in_specs=[pl.BlockSpec((_B // 16, _T, _D), lambda i: (i, 0, 0))],
        out_specs=pl.BlockSpec((_B // 16, _D), lambda i: (i, 0)),
        out_shape=jax.ShapeDtypeStruct((_B, _D), jnp.float32),
    )(x_embed)

    sim = pl.pallas_call(
        _sim_body,
        grid=(8,),
        in_specs=[
            pl.BlockSpec((_B, _D), lambda i: (0, 0)),
            pl.BlockSpec((_P // 8, _D),
                         lambda i: (jnp.minimum(i, 3), 0)),
            pl.BlockSpec((_P // 8, _D),
                         lambda i: (jnp.maximum(i - 4, 0), 0)),
        ],
        out_specs=pl.BlockSpec((_B, _P // 8), lambda i: (0, i)),
        out_shape=jax.ShapeDtypeStruct((_B, _P), jnp.float32),
    )(q_norm, prompt_key_old, prompt_key_new)

    idx, row, acc = pl.pallas_call(
        _topk_body,
        grid=(4,),
        in_specs=[pl.BlockSpec((_B // 4, _P), lambda i: (i, 0))],
        out_specs=[
            pl.BlockSpec((_B // 4, _K), lambda i: (i, 0)),
            pl.BlockSpec((_B // 4, _KL), lambda i: (i, 0)),
            pl.BlockSpec((1, 1, 128), lambda i: (i, 0, 0)),
        ],
        out_shape=[
            jax.ShapeDtypeStruct((_B, _K), jnp.int32),
            jax.ShapeDtypeStruct((_B, _KL), jnp.int32),
            jax.ShapeDtypeStruct((4, 1, 128), jnp.float32),
        ],
    )(sim)

    out2d = _sc_gather()(prompt2d, row.reshape(-1),
                         x_embed.reshape(_B * _T, _D))
    prompted = out2d.reshape(_B, _OUT_ROWS, _D)
    reduce_sim = jnp.sum(acc[:, 0, 0]) / _B
    return prompted, reduce_sim, sim, idx
